# baseline (speedup 1.0000x reference)
import jax
import jax.numpy as jnp
from jax import lax
from jax.experimental import pallas as pl
from jax.experimental.pallas import tpu as pltpu

BATCH = 16384
NUM_ACTIONS = 1000
NSENS = 16
NGROUPS = 10
LAMBDA_FAIRNESS = 0.01


def _pairmask():
    rj = lax.broadcasted_iota(jnp.int32, (16, 16), 0)
    ck = lax.broadcasted_iota(jnp.int32, (16, 16), 1)
    same = jnp.logical_or(
        jnp.logical_and(rj < 2, ck < 2),
        jnp.logical_and(jnp.logical_and(rj >= 2, rj < 10),
                        jnp.logical_and(ck >= 2, ck < 10)))
    return jnp.where(jnp.logical_and(rj > ck, same), 1.0, 0.0)


def _body(x_ref, g_ref, a_ref, out_ref):
    XT = x_ref[...]
    gvec = g_ref[...]
    avec = a_ref[...]
    gi = lax.broadcasted_iota(jnp.int32, (16, BATCH), 0)
    is_gender = gi < 2
    oh_bool = jnp.logical_or(
        jnp.logical_and(is_gender, gvec == gi),
        jnp.logical_and(
            jnp.logical_and(jnp.logical_not(is_gender), gi < NGROUPS),
            avec == (gi - 2)))
    MT = jnp.where(oh_bool, 1.0, 0.0)

    sums = lax.dot_general(MT, XT, (((1,), (1,)), ((), ())),
                           preferred_element_type=jnp.float32)
    counts = lax.dot_general(MT, jnp.ones((1, BATCH), jnp.float32),
                             (((1,), (1,)), ((), ())),
                             preferred_element_type=jnp.float32)

    present = jnp.where(counts > 0.0, 1.0, 0.0)
    safe = jnp.where(counts > 0.0, counts, 1.0)
    means = sums / safe
    both = lax.dot_general(present, present, (((1,), (1,)), ((), ())),
                           preferred_element_type=jnp.float32)
    pm = _pairmask()
    ncomp = float(NSENS) * jnp.sum(pm * both)
    total = jnp.float32(0.0)
    for k in range(NGROUPS):
        d = jnp.abs(means - means[k:k + 1, :])
        s = jnp.sum(d, axis=1, keepdims=True)
        total = total + jnp.sum(s * pm[:, k:k + 1] * both[:, k:k + 1])
    result = jnp.where(
        ncomp > 0.0,
        LAMBDA_FAIRNESS * total / jnp.maximum(ncomp, 1.0),
        0.0)
    out_ref[0, 0] = result


@jax.jit
def kernel(action_probs, demo_gender, demo_age):
    xt = action_probs[:, :NSENS].T
    out = pl.pallas_call(
        _body,
        out_specs=pl.BlockSpec(memory_space=pltpu.SMEM),
        out_shape=jax.ShapeDtypeStruct((1, 1), jnp.float32),
        compiler_params=pltpu.CompilerParams(
            allow_input_fusion=[True, True, True]),
    )(xt, demo_gender.reshape(1, BATCH), demo_age.reshape(1, BATCH))
    return out[0, 0]
